# baseline (speedup 1.0000x reference)
import functools

import jax
import jax.numpy as jnp
from jax import lax
from jax.experimental import pallas as pl
from jax.experimental.pallas import tpu as pltpu
from jax.experimental.pallas import tpu_sc as plsc

V = 2048
D = 256
E = 262144
M = 128
H = 8
C = 32
HC = H * C

NC = 2
NS = 16
NW = NC * NS
EK = 64
NCHUNK = E // (NW * EK)
LANES = 16

_NEG_SLOPE = 0.2


def _pre_body(x_ref, mat_ref, Ws_ref, bs_ref, Wl1_ref, Wl2_ref, bl_ref,
              Wr1_ref, Wr2_ref, br_ref, xl_ref, xr_ref):
    x = x_ref[...]
    mat = mat_ref[...]
    xs = lax.dot_general(mat, x, (((0,), (0,)), ((), ())),
                         preferred_element_type=jnp.float32)
    spec = jnp.dot(mat, jnp.dot(xs, Ws_ref[...]) + bs_ref[...],
                   preferred_element_type=jnp.float32)
    xl_ref[...] = (jnp.dot(x, Wl1_ref[...], preferred_element_type=jnp.float32)
                   + jnp.dot(spec, Wl2_ref[...], preferred_element_type=jnp.float32)
                   + bl_ref[...])
    xr_ref[...] = (jnp.dot(x, Wr1_ref[...], preferred_element_type=jnp.float32)
                   + jnp.dot(spec, Wr2_ref[...], preferred_element_type=jnp.float32)
                   + br_ref[...])


def _tc_pre(x, mat, Ws, bs, Wl1, Wl2, bl, Wr1, Wr2, br):
    return pl.pallas_call(
        _pre_body,
        out_shape=(jax.ShapeDtypeStruct((V, HC), jnp.float32),
                   jax.ShapeDtypeStruct((V, HC), jnp.float32)),
    )(x, mat, Ws, bs, Wl1, Wl2, bl, Wr1, Wr2, br)


def _sc_body(xl_hbm, xr_hbm, src_hbm, dst_hbm, ea_hbm, we_hbm, att_hbm,
             agg_out, asum_out,
             src_v, dst_v, ea_v, xl_stage, xr_stage, agg_stage, asum_stage,
             wea_v, ea_splat, sem0, sem1, sem2, sem3, agg_sh, asum_sh):
    cid = lax.axis_index("c")
    sid = lax.axis_index("s")
    wid = cid * NS + sid

    pltpu.sync_copy(src_hbm.at[pl.ds(wid * NCHUNK, NCHUNK)], src_v)
    pltpu.sync_copy(dst_hbm.at[pl.ds(wid * NCHUNK, NCHUNK)], dst_v)
    pltpu.sync_copy(ea_hbm.at[pl.ds(wid * NCHUNK, NCHUNK)], ea_v)

    pltpu.sync_copy(we_hbm, wea_v.at[0])
    pltpu.sync_copy(att_hbm, wea_v.at[1])

    zero16 = jnp.zeros((LANES,), jnp.float32)

    def _zero_row(r, _):
        for c in range(HC // LANES):
            agg_stage[0, r, pl.ds(c * LANES, LANES)] = zero16
        asum_stage[0, r, :] = zero16
        asum_stage[1, r, :] = zero16
        return _
    lax.fori_loop(0, EK, _zero_row, None)

    rows_per_sid = V // NS
    for b in range(rows_per_sid // EK):
        pltpu.sync_copy(agg_stage.at[0],
                        agg_sh.at[pl.ds(sid * rows_per_sid + b * EK, EK)])
    for b in range(rows_per_sid // EK):
        pltpu.sync_copy(asum_stage.at[0],
                        asum_sh.at[pl.ds(sid * rows_per_sid + b * EK, EK)])
    plsc.subcore_barrier()

    lane_iota = lax.iota(jnp.int32, LANES)
    zero16f = jnp.zeros((LANES,), jnp.float32)
    mask8 = lane_iota < H

    def _start_gather(j, slot):
        pltpu.async_copy(xl_hbm.at[src_v.at[j]], xl_stage.at[slot], sem0)
        pltpu.async_copy(xr_hbm.at[dst_v.at[j]], xr_stage.at[slot], sem1)

    def _wait_gather(j, slot):
        pltpu.make_async_copy(xl_hbm.at[src_v.at[j]], xl_stage.at[slot],
                              sem0).wait()
        pltpu.make_async_copy(xr_hbm.at[dst_v.at[j]], xr_stage.at[slot],
                              sem1).wait()

    def _start_scatter(j, slot):
        pltpu.async_copy(agg_stage.at[slot], agg_sh.at[dst_v.at[j]], sem2,
                         add=True)
        pltpu.async_copy(asum_stage.at[slot], asum_sh.at[dst_v.at[j]], sem3,
                         add=True)

    def _wait_scatter(j, slot):
        pltpu.make_async_copy(agg_stage.at[slot], agg_sh.at[dst_v.at[j]],
                              sem2).wait()
        pltpu.make_async_copy(asum_stage.at[slot], asum_sh.at[dst_v.at[j]],
                              sem3).wait()

    _start_gather(0, 0)

    def _chunk(j, _):
        slot = lax.rem(j, 2)
        _wait_gather(j, slot)
        pl.when(j < NCHUNK - 1)(lambda: _start_gather(j + 1, 1 - slot))
        pl.when(j >= 2)(lambda: _wait_scatter(j - 2, slot))

        xl_b = xl_stage.at[slot]
        xr_b = xr_stage.at[slot]
        agg_b = agg_stage.at[slot]
        asum_b = asum_stage.at[slot]

        for g in range(EK // LANES):
            ea16 = ea_v[j, pl.ds(g * LANES, LANES)]
            for i in range(LANES):
                sp = jnp.broadcast_to(ea16[i], (LANES,))
                ea_splat[g * LANES + i, :] = plsc.pack(
                    sp, sp, format=plsc.PackFormat.INTERLEAVED)

        def _unp(x):
            return plsc.unpack(x, format=plsc.PackFormat.INTERLEAVED,
                               preferred_element_type=jnp.float32)

        neg_slope_bf = jnp.bfloat16(_NEG_SLOPE)

        @plsc.parallel_loop(0, EK, 1, unroll=2)
        def _edge(e):
            eav = ea_splat[e, :]
            rs = []
            for h in range(H):
                a = xl_b[e, pl.ds(h * C, C)]
                b = xr_b[e, pl.ds(h * C, C)]
                u = a + b + eav * wea_v[0, pl.ds(h * C, C)]
                t = jnp.maximum(u, neg_slope_bf * u)
                s0, s1 = _unp(t * wea_v[1, pl.ds(h * C, C)])
                rs.append(jnp.sum(s0 + s1))
            parts = [jnp.where(lane_iota == h, rs[h], 0.0) for h in range(H)]
            av = ((parts[0] + parts[1]) + (parts[2] + parts[3])) + (
                (parts[4] + parts[5]) + (parts[6] + parts[7]))
            w = jnp.where(mask8, jnp.exp(av), 0.0)
            asum_b[e, :] = w

        @plsc.parallel_loop(0, EK, 1, unroll=1)
        def _edge_agg(e):
            w = asum_b[e, :]
            for h in range(H):
                a0, a1 = _unp(xl_b[e, pl.ds(h * C, C)])
                wh = w[h]
                agg_b[e, pl.ds(h * C, LANES)] = wh * a0
                agg_b[e, pl.ds(h * C + LANES, LANES)] = wh * a1

        _start_scatter(j, slot)
        return _
    lax.fori_loop(0, NCHUNK, _chunk, None)

    _wait_scatter(NCHUNK - 2, 0)
    _wait_scatter(NCHUNK - 1, 1)
    plsc.subcore_barrier()
    out_row = cid * V + sid * rows_per_sid
    pltpu.sync_copy(agg_sh.at[pl.ds(sid * rows_per_sid, rows_per_sid)],
                    agg_out.at[pl.ds(out_row, rows_per_sid)])
    pltpu.sync_copy(asum_sh.at[pl.ds(sid * rows_per_sid, rows_per_sid)],
                    asum_out.at[pl.ds(out_row, rows_per_sid)])


def _sc_edge(xl, xr, src2d, dst2d, ea2d, we_flat, att_flat):
    mesh = plsc.VectorSubcoreMesh(core_axis_name="c", subcore_axis_name="s")
    kfn = pl.kernel(
        _sc_body,
        out_type=(jax.ShapeDtypeStruct((NC * V, HC), jnp.float32),
                  jax.ShapeDtypeStruct((NC * V, LANES), jnp.float32)),
        mesh=mesh,
        compiler_params=pltpu.CompilerParams(use_tc_tiling_on_sc=False,
                                             needs_layout_passes=False),
        scratch_types=[
            pltpu.VMEM((NCHUNK, EK), jnp.int32),
            pltpu.VMEM((NCHUNK, EK), jnp.int32),
            pltpu.VMEM((NCHUNK, EK), jnp.float32),
            pltpu.VMEM((2, EK, HC), jnp.bfloat16),
            pltpu.VMEM((2, EK, HC), jnp.bfloat16),
            pltpu.VMEM((2, EK, HC), jnp.float32),
            pltpu.VMEM((2, EK, LANES), jnp.float32),
            pltpu.VMEM((2, HC), jnp.bfloat16),
            pltpu.VMEM((EK, C), jnp.bfloat16),
            pltpu.SemaphoreType.DMA,
            pltpu.SemaphoreType.DMA,
            pltpu.SemaphoreType.DMA,
            pltpu.SemaphoreType.DMA,
            pltpu.VMEM_SHARED((V, HC), jnp.float32),
            pltpu.VMEM_SHARED((V, LANES), jnp.float32),
        ],
    )
    return kfn(xl, xr, src2d, dst2d, ea2d, we_flat, att_flat)


def _graph_norm(y, w, b):
    m = jnp.mean(y)
    s = jnp.sqrt(jnp.mean((y - m) * (y - m)))
    return (y - m) / (s + 1e-5) * w + b


def _post_body(x_ref, xl_ref, xr_ref, ei_ref, agg_ref, asum_ref, we_ref,
               att_ref, gb_ref, n1w_ref, n1b_ref, W2_ref, b2_ref, n2w_ref,
               n2b_ref, W3_ref, b3_ref, n3w_ref, n3b_ref, out_ref):
    x = x_ref[...]
    xl = xl_ref[...]
    xr = xr_ref[...]
    mea = jnp.mean(ei_ref[...])

    row = lax.broadcasted_iota(jnp.int32, (LANES, HC), 0)
    col = lax.broadcasted_iota(jnp.int32, (LANES, HC), 1)
    B = (col // C == row).astype(jnp.float32)

    u = xl + xr + mea * we_ref[...]
    t = jnp.maximum(u, _NEG_SLOPE * u)
    alpha_l = lax.dot_general(t * att_ref[...], B, (((1,), (1,)), ((), ())),
                              preferred_element_type=jnp.float32)
    w_l = jnp.exp(alpha_l)
    asum = asum_ref[0] + asum_ref[1] + w_l
    num = agg_ref[0] + agg_ref[1] + jnp.dot(
        w_l, B, preferred_element_type=jnp.float32) * xl
    recip = 1.0 / (asum + 1e-16)
    gat = num * jnp.dot(recip, B, preferred_element_type=jnp.float32) + gb_ref[...]

    x1 = _graph_norm(x + jnp.maximum(gat, 0.0), n1w_ref[...], n1b_ref[...])
    x2 = _graph_norm(
        x1 + jnp.dot(x1, W2_ref[...], preferred_element_type=jnp.float32)
        + b2_ref[...], n2w_ref[...], n2b_ref[...])
    x3 = _graph_norm(
        x2 + jnp.dot(x2, W3_ref[...], preferred_element_type=jnp.float32)
        + b3_ref[...], n3w_ref[...], n3b_ref[...])
    out_ref[...] = x3


def _tc_post(x, xl, xr, ei2d, agg_p, asum_p, we_row, att_row, gat_bias,
             n1w, n1b, W2, b2, n2w, n2b, W3, b3, n3w, n3b):
    return pl.pallas_call(
        _post_body,
        out_shape=jax.ShapeDtypeStruct((V, D), jnp.float32),
    )(x, xl, xr, ei2d, agg_p, asum_p, we_row, att_row, gat_bias,
      n1w, n1b, W2, b2, n2w, n2b, W3, b3, n3w, n3b)


@jax.jit
def kernel(x, full_edge_index, edge_indicators, batch, spectrum_mats, Ws, bs,
           Wl, bl, Wr, br, We, att, gat_bias, n1w, n1b, W2, b2, n2w, n2b,
           W3, b3, n3w, n3b):
    mat = spectrum_mats[0]
    xl, xr = _tc_pre(x, mat, Ws, bs, Wl[:D], Wl[D:], bl, Wr[:D], Wr[D:], br)

    src2d = full_edge_index[0].reshape(NW * NCHUNK, EK)
    dst2d = full_edge_index[1].reshape(NW * NCHUNK, EK)
    ea2d = edge_indicators.reshape(NW * NCHUNK, EK)
    we_flat = We.reshape(HC)
    att_flat = att.reshape(HC)

    def _ilv(t):
        n = t.shape[0]
        return (t.reshape(n, H, 2, LANES).transpose(0, 1, 3, 2)
                .reshape(n, HC).astype(jnp.bfloat16))

    agg_p, asum_p = _sc_edge(_ilv(xl), _ilv(xr), src2d, dst2d, ea2d,
                             _ilv(we_flat.reshape(1, HC)).reshape(HC),
                             _ilv(att_flat.reshape(1, HC)).reshape(HC))
    agg_p = agg_p.reshape(NC, V, HC)
    asum_p = asum_p.reshape(NC, V, LANES)

    ei2d = edge_indicators.reshape(V, E // V)
    return _tc_post(x, xl, xr, ei2d, agg_p, asum_p, We.reshape(1, HC),
                    att.reshape(1, HC), gat_bias, n1w, n1b, W2, b2, n2w,
                    n2b, W3, b3, n3w, n3b)


# speedup vs baseline: 1.0016x; 1.0016x over previous
import functools

import jax
import jax.numpy as jnp
from jax import lax
from jax.experimental import pallas as pl
from jax.experimental.pallas import tpu as pltpu
from jax.experimental.pallas import tpu_sc as plsc

V = 2048
D = 256
E = 262144
M = 128
H = 8
C = 32
HC = H * C

NC = 2
NS = 16
NW = NC * NS
EK = 64
NCHUNK = E // (NW * EK)
LANES = 16

_NEG_SLOPE = 0.2


def _pre_body(x_ref, mat_ref, Ws_ref, bs_ref, Wl1_ref, Wl2_ref, bl_ref,
              Wr1_ref, Wr2_ref, br_ref, xl_ref, xr_ref):
    x = x_ref[...]
    mat = mat_ref[...]
    xs = lax.dot_general(mat, x, (((0,), (0,)), ((), ())),
                         preferred_element_type=jnp.float32)
    spec = jnp.dot(mat, jnp.dot(xs, Ws_ref[...]) + bs_ref[...],
                   preferred_element_type=jnp.float32)
    xl_ref[...] = (jnp.dot(x, Wl1_ref[...], preferred_element_type=jnp.float32)
                   + jnp.dot(spec, Wl2_ref[...], preferred_element_type=jnp.float32)
                   + bl_ref[...])
    xr_ref[...] = (jnp.dot(x, Wr1_ref[...], preferred_element_type=jnp.float32)
                   + jnp.dot(spec, Wr2_ref[...], preferred_element_type=jnp.float32)
                   + br_ref[...])


def _tc_pre(x, mat, Ws, bs, Wl1, Wl2, bl, Wr1, Wr2, br):
    return pl.pallas_call(
        _pre_body,
        out_shape=(jax.ShapeDtypeStruct((V, HC), jnp.float32),
                   jax.ShapeDtypeStruct((V, HC), jnp.float32)),
    )(x, mat, Ws, bs, Wl1, Wl2, bl, Wr1, Wr2, br)


def _sc_body(xl_hbm, xr_hbm, src_hbm, dst_hbm, ea_hbm, we_hbm, att_hbm,
             agg_out, asum_out,
             src_v, dst_v, ea_v, xl_stage, xr_stage, agg_stage, asum_stage,
             wea_v, ea_splat, sem0, sem1, sem2, sem3, agg_sh, asum_sh):
    cid = lax.axis_index("c")
    sid = lax.axis_index("s")
    wid = cid * NS + sid

    pltpu.sync_copy(src_hbm.at[pl.ds(wid * NCHUNK, NCHUNK)], src_v)
    pltpu.sync_copy(dst_hbm.at[pl.ds(wid * NCHUNK, NCHUNK)], dst_v)
    pltpu.sync_copy(ea_hbm.at[pl.ds(wid * NCHUNK, NCHUNK)], ea_v)

    pltpu.sync_copy(we_hbm, wea_v.at[0])
    pltpu.sync_copy(att_hbm, wea_v.at[1])

    zero16 = jnp.zeros((LANES,), jnp.float32)

    def _zero_row(r, _):
        for c in range(HC // LANES):
            agg_stage[0, r, pl.ds(c * LANES, LANES)] = zero16
        asum_stage[0, r, :] = zero16
        asum_stage[1, r, :] = zero16
        return _
    lax.fori_loop(0, EK, _zero_row, None)

    rows_per_sid = V // NS
    for b in range(rows_per_sid // EK):
        pltpu.sync_copy(agg_stage.at[0],
                        agg_sh.at[pl.ds(sid * rows_per_sid + b * EK, EK)])
    for b in range(rows_per_sid // EK):
        pltpu.sync_copy(asum_stage.at[0],
                        asum_sh.at[pl.ds(sid * rows_per_sid + b * EK, EK)])
    plsc.subcore_barrier()

    lane_iota = lax.iota(jnp.int32, LANES)
    zero16f = jnp.zeros((LANES,), jnp.float32)
    mask8 = lane_iota < H

    def _start_gather(j, slot):
        pltpu.async_copy(xl_hbm.at[src_v.at[j]], xl_stage.at[slot], sem0)
        pltpu.async_copy(xr_hbm.at[dst_v.at[j]], xr_stage.at[slot], sem1)

    def _wait_gather(j, slot):
        pltpu.make_async_copy(xl_hbm.at[src_v.at[j]], xl_stage.at[slot],
                              sem0).wait()
        pltpu.make_async_copy(xr_hbm.at[dst_v.at[j]], xr_stage.at[slot],
                              sem1).wait()

    def _start_scatter(j, slot):
        pltpu.async_copy(agg_stage.at[slot], agg_sh.at[dst_v.at[j]], sem2,
                         add=True)
        pltpu.async_copy(asum_stage.at[slot], asum_sh.at[dst_v.at[j]], sem3,
                         add=True)

    def _wait_scatter(j, slot):
        pltpu.make_async_copy(agg_stage.at[slot], agg_sh.at[dst_v.at[j]],
                              sem2).wait()
        pltpu.make_async_copy(asum_stage.at[slot], asum_sh.at[dst_v.at[j]],
                              sem3).wait()

    _start_gather(0, 0)

    def _chunk(j, _):
        slot = lax.rem(j, 2)
        _wait_gather(j, slot)
        pl.when(j < NCHUNK - 1)(lambda: _start_gather(j + 1, 1 - slot))
        pl.when(j >= 2)(lambda: _wait_scatter(j - 2, slot))

        xl_b = xl_stage.at[slot]
        xr_b = xr_stage.at[slot]
        agg_b = agg_stage.at[slot]
        asum_b = asum_stage.at[slot]

        for g in range(EK // LANES):
            ea16 = ea_v[j, pl.ds(g * LANES, LANES)]
            for i in range(LANES):
                sp = jnp.broadcast_to(ea16[i], (LANES,))
                ea_splat[g * LANES + i, :] = plsc.pack(
                    sp, sp, format=plsc.PackFormat.INTERLEAVED)

        def _unp(x):
            return plsc.unpack(x, format=plsc.PackFormat.INTERLEAVED,
                               preferred_element_type=jnp.float32)

        neg_slope_bf = jnp.bfloat16(_NEG_SLOPE)

        @plsc.parallel_loop(0, EK, 1, unroll=3)
        def _edge(e):
            eav = ea_splat[e, :]
            rs = []
            for h in range(H):
                a = xl_b[e, pl.ds(h * C, C)]
                b = xr_b[e, pl.ds(h * C, C)]
                u = a + b + eav * wea_v[0, pl.ds(h * C, C)]
                t = jnp.maximum(u, neg_slope_bf * u)
                s0, s1 = _unp(t * wea_v[1, pl.ds(h * C, C)])
                rs.append(jnp.sum(s0 + s1))
            parts = [jnp.where(lane_iota == h, rs[h], 0.0) for h in range(H)]
            av = ((parts[0] + parts[1]) + (parts[2] + parts[3])) + (
                (parts[4] + parts[5]) + (parts[6] + parts[7]))
            w = jnp.where(mask8, jnp.exp(av), 0.0)
            asum_b[e, :] = w

        @plsc.parallel_loop(0, EK, 1, unroll=2)
        def _edge_agg(e):
            w = asum_b[e, :]
            for h in range(H):
                a0, a1 = _unp(xl_b[e, pl.ds(h * C, C)])
                wh = w[h]
                agg_b[e, pl.ds(h * C, LANES)] = wh * a0
                agg_b[e, pl.ds(h * C + LANES, LANES)] = wh * a1

        _start_scatter(j, slot)
        return _
    lax.fori_loop(0, NCHUNK, _chunk, None)

    _wait_scatter(NCHUNK - 2, 0)
    _wait_scatter(NCHUNK - 1, 1)
    plsc.subcore_barrier()
    out_row = cid * V + sid * rows_per_sid
    pltpu.sync_copy(agg_sh.at[pl.ds(sid * rows_per_sid, rows_per_sid)],
                    agg_out.at[pl.ds(out_row, rows_per_sid)])
    pltpu.sync_copy(asum_sh.at[pl.ds(sid * rows_per_sid, rows_per_sid)],
                    asum_out.at[pl.ds(out_row, rows_per_sid)])


def _sc_edge(xl, xr, src2d, dst2d, ea2d, we_flat, att_flat):
    mesh = plsc.VectorSubcoreMesh(core_axis_name="c", subcore_axis_name="s")
    kfn = pl.kernel(
        _sc_body,
        out_type=(jax.ShapeDtypeStruct((NC * V, HC), jnp.float32),
                  jax.ShapeDtypeStruct((NC * V, LANES), jnp.float32)),
        mesh=mesh,
        compiler_params=pltpu.CompilerParams(use_tc_tiling_on_sc=False,
                                             needs_layout_passes=False),
        scratch_types=[
            pltpu.VMEM((NCHUNK, EK), jnp.int32),
            pltpu.VMEM((NCHUNK, EK), jnp.int32),
            pltpu.VMEM((NCHUNK, EK), jnp.float32),
            pltpu.VMEM((2, EK, HC), jnp.bfloat16),
            pltpu.VMEM((2, EK, HC), jnp.bfloat16),
            pltpu.VMEM((2, EK, HC), jnp.float32),
            pltpu.VMEM((2, EK, LANES), jnp.float32),
            pltpu.VMEM((2, HC), jnp.bfloat16),
            pltpu.VMEM((EK, C), jnp.bfloat16),
            pltpu.SemaphoreType.DMA,
            pltpu.SemaphoreType.DMA,
            pltpu.SemaphoreType.DMA,
            pltpu.SemaphoreType.DMA,
            pltpu.VMEM_SHARED((V, HC), jnp.float32),
            pltpu.VMEM_SHARED((V, LANES), jnp.float32),
        ],
    )
    return kfn(xl, xr, src2d, dst2d, ea2d, we_flat, att_flat)


def _graph_norm(y, w, b):
    m = jnp.mean(y)
    s = jnp.sqrt(jnp.mean((y - m) * (y - m)))
    return (y - m) / (s + 1e-5) * w + b


def _post_body(x_ref, xl_ref, xr_ref, ei_ref, agg_ref, asum_ref, we_ref,
               att_ref, gb_ref, n1w_ref, n1b_ref, W2_ref, b2_ref, n2w_ref,
               n2b_ref, W3_ref, b3_ref, n3w_ref, n3b_ref, out_ref):
    x = x_ref[...]
    xl = xl_ref[...]
    xr = xr_ref[...]
    mea = jnp.mean(ei_ref[...])

    row = lax.broadcasted_iota(jnp.int32, (LANES, HC), 0)
    col = lax.broadcasted_iota(jnp.int32, (LANES, HC), 1)
    B = (col // C == row).astype(jnp.float32)

    u = xl + xr + mea * we_ref[...]
    t = jnp.maximum(u, _NEG_SLOPE * u)
    alpha_l = lax.dot_general(t * att_ref[...], B, (((1,), (1,)), ((), ())),
                              preferred_element_type=jnp.float32)
    w_l = jnp.exp(alpha_l)
    asum = asum_ref[0] + asum_ref[1] + w_l
    num = agg_ref[0] + agg_ref[1] + jnp.dot(
        w_l, B, preferred_element_type=jnp.float32) * xl
    recip = 1.0 / (asum + 1e-16)
    gat = num * jnp.dot(recip, B, preferred_element_type=jnp.float32) + gb_ref[...]

    x1 = _graph_norm(x + jnp.maximum(gat, 0.0), n1w_ref[...], n1b_ref[...])
    x2 = _graph_norm(
        x1 + jnp.dot(x1, W2_ref[...], preferred_element_type=jnp.float32)
        + b2_ref[...], n2w_ref[...], n2b_ref[...])
    x3 = _graph_norm(
        x2 + jnp.dot(x2, W3_ref[...], preferred_element_type=jnp.float32)
        + b3_ref[...], n3w_ref[...], n3b_ref[...])
    out_ref[...] = x3


def _tc_post(x, xl, xr, ei2d, agg_p, asum_p, we_row, att_row, gat_bias,
             n1w, n1b, W2, b2, n2w, n2b, W3, b3, n3w, n3b):
    return pl.pallas_call(
        _post_body,
        out_shape=jax.ShapeDtypeStruct((V, D), jnp.float32),
    )(x, xl, xr, ei2d, agg_p, asum_p, we_row, att_row, gat_bias,
      n1w, n1b, W2, b2, n2w, n2b, W3, b3, n3w, n3b)


@jax.jit
def kernel(x, full_edge_index, edge_indicators, batch, spectrum_mats, Ws, bs,
           Wl, bl, Wr, br, We, att, gat_bias, n1w, n1b, W2, b2, n2w, n2b,
           W3, b3, n3w, n3b):
    mat = spectrum_mats[0]
    xl, xr = _tc_pre(x, mat, Ws, bs, Wl[:D], Wl[D:], bl, Wr[:D], Wr[D:], br)

    src2d = full_edge_index[0].reshape(NW * NCHUNK, EK)
    dst2d = full_edge_index[1].reshape(NW * NCHUNK, EK)
    ea2d = edge_indicators.reshape(NW * NCHUNK, EK)
    we_flat = We.reshape(HC)
    att_flat = att.reshape(HC)

    def _ilv(t):
        n = t.shape[0]
        return (t.reshape(n, H, 2, LANES).transpose(0, 1, 3, 2)
                .reshape(n, HC).astype(jnp.bfloat16))

    agg_p, asum_p = _sc_edge(_ilv(xl), _ilv(xr), src2d, dst2d, ea2d,
                             _ilv(we_flat.reshape(1, HC)).reshape(HC),
                             _ilv(att_flat.reshape(1, HC)).reshape(HC))
    agg_p = agg_p.reshape(NC, V, HC)
    asum_p = asum_p.reshape(NC, V, LANES)

    ei2d = edge_indicators.reshape(V, E // V)
    return _tc_post(x, xl, xr, ei2d, agg_p, asum_p, We.reshape(1, HC),
                    att.reshape(1, HC), gat_bias, n1w, n1b, W2, b2, n2w,
                    n2b, W3, b3, n3w, n3b)


# speedup vs baseline: 1.0040x; 1.0024x over previous
import jax
import jax.numpy as jnp
from jax import lax
from jax.experimental import pallas as pl
from jax.experimental.pallas import tpu as pltpu
from jax.experimental.pallas import tpu_sc as plsc

V = 2048
D = 256
E = 262144
M = 128
H = 8
C = 32
HC = H * C

NC = 2
NS = 16
NW = NC * NS
EK = 64
NCHUNK = E // (NW * EK)
LANES = 16

_NEG_SLOPE = 0.2


def _pre_body(x_ref, mat_ref, Ws_ref, bs_ref, Wl1_ref, Wl2_ref, bl_ref,
              Wr1_ref, Wr2_ref, br_ref, xl_ref, xr_ref):
    x = x_ref[...]
    mat = mat_ref[...]
    xs = lax.dot_general(mat, x, (((0,), (0,)), ((), ())),
                         preferred_element_type=jnp.float32)
    spec = jnp.dot(mat, jnp.dot(xs, Ws_ref[...]) + bs_ref[...],
                   preferred_element_type=jnp.float32)
    xl_ref[...] = (jnp.dot(x, Wl1_ref[...], preferred_element_type=jnp.float32)
                   + jnp.dot(spec, Wl2_ref[...], preferred_element_type=jnp.float32)
                   + bl_ref[...])
    xr_ref[...] = (jnp.dot(x, Wr1_ref[...], preferred_element_type=jnp.float32)
                   + jnp.dot(spec, Wr2_ref[...], preferred_element_type=jnp.float32)
                   + br_ref[...])


def _tc_pre(x, mat, Ws, bs, Wl1, Wl2, bl, Wr1, Wr2, br):
    return pl.pallas_call(
        _pre_body,
        out_shape=(jax.ShapeDtypeStruct((V, HC), jnp.float32),
                   jax.ShapeDtypeStruct((V, HC), jnp.float32)),
    )(x, mat, Ws, bs, Wl1, Wl2, bl, Wr1, Wr2, br)


def _sc_body(xl_hbm, xr_hbm, src_hbm, dst_hbm, ea_hbm, we_hbm, att_hbm,
             agg_out, asum_out,
             src_v, dst_v, ea_v, xl_stage, xr_stage, agg_stage, asum_stage,
             wea_v, ea_splat, sem0, sem1, sem2, sem3, agg_sh, asum_sh):
    cid = lax.axis_index("c")
    sid = lax.axis_index("s")
    wid = cid * NS + sid

    pltpu.sync_copy(src_hbm.at[pl.ds(wid * NCHUNK, NCHUNK)], src_v)
    pltpu.sync_copy(dst_hbm.at[pl.ds(wid * NCHUNK, NCHUNK)], dst_v)
    pltpu.sync_copy(ea_hbm.at[pl.ds(wid * NCHUNK, NCHUNK)], ea_v)

    pltpu.sync_copy(we_hbm, wea_v.at[0])
    pltpu.sync_copy(att_hbm, wea_v.at[1])

    zero16 = jnp.zeros((LANES,), jnp.float32)

    def _zero_row(r, _):
        for c in range(HC // LANES):
            agg_stage[0, r, pl.ds(c * LANES, LANES)] = zero16
        asum_stage[0, r, :] = zero16
        asum_stage[1, r, :] = zero16
        return _
    lax.fori_loop(0, EK, _zero_row, None)

    rows_per_sid = V // NS
    for b in range(rows_per_sid // EK):
        pltpu.sync_copy(agg_stage.at[0],
                        agg_sh.at[pl.ds(sid * rows_per_sid + b * EK, EK)])
    for b in range(rows_per_sid // EK):
        pltpu.sync_copy(asum_stage.at[0],
                        asum_sh.at[pl.ds(sid * rows_per_sid + b * EK, EK)])
    plsc.subcore_barrier()

    lane_iota = lax.iota(jnp.int32, LANES)
    mask8 = lane_iota < H

    def _start_gather(j, slot):
        pltpu.async_copy(xl_hbm.at[src_v.at[j]], xl_stage.at[slot], sem0)
        pltpu.async_copy(xr_hbm.at[dst_v.at[j]], xr_stage.at[slot], sem1)

    def _wait_gather(j, slot):
        pltpu.make_async_copy(xl_hbm.at[src_v.at[j]], xl_stage.at[slot],
                              sem0).wait()
        pltpu.make_async_copy(xr_hbm.at[dst_v.at[j]], xr_stage.at[slot],
                              sem1).wait()

    def _start_scatter(j, slot):
        pltpu.async_copy(agg_stage.at[slot], agg_sh.at[dst_v.at[j]], sem2,
                         add=True)
        pltpu.async_copy(asum_stage.at[slot], asum_sh.at[dst_v.at[j]], sem3,
                         add=True)

    def _wait_scatter(j, slot):
        pltpu.make_async_copy(agg_stage.at[slot], agg_sh.at[dst_v.at[j]],
                              sem2).wait()
        pltpu.make_async_copy(asum_stage.at[slot], asum_sh.at[dst_v.at[j]],
                              sem3).wait()

    _start_gather(0, 0)

    def _chunk(j, _):
        slot = lax.rem(j, 2)
        _wait_gather(j, slot)
        pl.when(j < NCHUNK - 1)(lambda: _start_gather(j + 1, 1 - slot))
        pl.when(j >= 2)(lambda: _wait_scatter(j - 2, slot))

        xl_b = xl_stage.at[slot]
        xr_b = xr_stage.at[slot]
        agg_b = agg_stage.at[slot]
        asum_b = asum_stage.at[slot]

        for g in range(EK // LANES):
            ea16 = ea_v[j, pl.ds(g * LANES, LANES)]
            for i in range(LANES):
                sp = jnp.broadcast_to(ea16[i], (LANES,))
                ea_splat[g * LANES + i, :] = plsc.pack(
                    sp, sp, format=plsc.PackFormat.INTERLEAVED)

        def _unp(x):
            return plsc.unpack(x, format=plsc.PackFormat.INTERLEAVED,
                               preferred_element_type=jnp.float32)

        neg_slope_bf = jnp.bfloat16(_NEG_SLOPE)

        @plsc.parallel_loop(0, EK, 1, unroll=2)
        def _edge(e):
            eav = ea_splat[e, :]
            rs = []
            for h in range(H):
                a = xl_b[e, pl.ds(h * C, C)]
                b = xr_b[e, pl.ds(h * C, C)]
                u = a + b + eav * wea_v[0, pl.ds(h * C, C)]
                t = jnp.maximum(u, neg_slope_bf * u)
                s0, s1 = _unp(t * wea_v[1, pl.ds(h * C, C)])
                rs.append(jnp.sum(s0 + s1))
            parts = [jnp.where(lane_iota == h, rs[h], 0.0) for h in range(H)]
            av = ((parts[0] + parts[1]) + (parts[2] + parts[3])) + (
                (parts[4] + parts[5]) + (parts[6] + parts[7]))
            w = jnp.where(mask8, jnp.exp(av), 0.0)
            asum_b[e, :] = w

        @plsc.parallel_loop(0, EK, 1, unroll=2)
        def _edge_agg(e):
            w = asum_b[e, :]
            for h in range(H):
                a0, a1 = _unp(xl_b[e, pl.ds(h * C, C)])
                wh = w[h]
                agg_b[e, pl.ds(h * C, LANES)] = wh * a0
                agg_b[e, pl.ds(h * C + LANES, LANES)] = wh * a1

        _start_scatter(j, slot)
        return _
    lax.fori_loop(0, NCHUNK, _chunk, None)

    _wait_scatter(NCHUNK - 2, 0)
    _wait_scatter(NCHUNK - 1, 1)
    plsc.subcore_barrier()
    out_row = cid * V + sid * rows_per_sid
    pltpu.sync_copy(agg_sh.at[pl.ds(sid * rows_per_sid, rows_per_sid)],
                    agg_out.at[pl.ds(out_row, rows_per_sid)])
    pltpu.sync_copy(asum_sh.at[pl.ds(sid * rows_per_sid, rows_per_sid)],
                    asum_out.at[pl.ds(out_row, rows_per_sid)])


def _sc_edge(xl, xr, src2d, dst2d, ea2d, we_flat, att_flat):
    mesh = plsc.VectorSubcoreMesh(core_axis_name="c", subcore_axis_name="s")
    kfn = pl.kernel(
        _sc_body,
        out_type=(jax.ShapeDtypeStruct((NC * V, HC), jnp.float32),
                  jax.ShapeDtypeStruct((NC * V, LANES), jnp.float32)),
        mesh=mesh,
        compiler_params=pltpu.CompilerParams(use_tc_tiling_on_sc=False,
                                             needs_layout_passes=False),
        scratch_types=[
            pltpu.VMEM((NCHUNK, EK), jnp.int32),
            pltpu.VMEM((NCHUNK, EK), jnp.int32),
            pltpu.VMEM((NCHUNK, EK), jnp.float32),
            pltpu.VMEM((2, EK, HC), jnp.bfloat16),
            pltpu.VMEM((2, EK, HC), jnp.bfloat16),
            pltpu.VMEM((2, EK, HC), jnp.float32),
            pltpu.VMEM((2, EK, LANES), jnp.float32),
            pltpu.VMEM((2, HC), jnp.bfloat16),
            pltpu.VMEM((EK, C), jnp.bfloat16),
            pltpu.SemaphoreType.DMA,
            pltpu.SemaphoreType.DMA,
            pltpu.SemaphoreType.DMA,
            pltpu.SemaphoreType.DMA,
            pltpu.VMEM_SHARED((V, HC), jnp.float32),
            pltpu.VMEM_SHARED((V, LANES), jnp.float32),
        ],
    )
    return kfn(xl, xr, src2d, dst2d, ea2d, we_flat, att_flat)


def _graph_norm(y, w, b):
    m = jnp.mean(y)
    s = jnp.sqrt(jnp.mean((y - m) * (y - m)))
    return (y - m) / (s + 1e-5) * w + b


def _post_body(x_ref, xl_ref, xr_ref, ei_ref, agg_ref, asum_ref, we_ref,
               att_ref, gb_ref, n1w_ref, n1b_ref, W2_ref, b2_ref, n2w_ref,
               n2b_ref, W3_ref, b3_ref, n3w_ref, n3b_ref, out_ref):
    x = x_ref[...]
    xl = xl_ref[...]
    xr = xr_ref[...]
    mea = jnp.mean(ei_ref[...])

    row = lax.broadcasted_iota(jnp.int32, (LANES, HC), 0)
    col = lax.broadcasted_iota(jnp.int32, (LANES, HC), 1)
    B = (col // C == row).astype(jnp.float32)

    u = xl + xr + mea * we_ref[...]
    t = jnp.maximum(u, _NEG_SLOPE * u)
    alpha_l = lax.dot_general(t * att_ref[...], B, (((1,), (1,)), ((), ())),
                              preferred_element_type=jnp.float32)
    w_l = jnp.exp(alpha_l)
    asum = asum_ref[0] + asum_ref[1] + w_l
    num = agg_ref[0] + agg_ref[1] + jnp.dot(
        w_l, B, preferred_element_type=jnp.float32) * xl
    recip = 1.0 / (asum + 1e-16)
    gat = num * jnp.dot(recip, B, preferred_element_type=jnp.float32) + gb_ref[...]

    x1 = _graph_norm(x + jnp.maximum(gat, 0.0), n1w_ref[...], n1b_ref[...])
    x2 = _graph_norm(
        x1 + jnp.dot(x1, W2_ref[...], preferred_element_type=jnp.float32)
        + b2_ref[...], n2w_ref[...], n2b_ref[...])
    x3 = _graph_norm(
        x2 + jnp.dot(x2, W3_ref[...], preferred_element_type=jnp.float32)
        + b3_ref[...], n3w_ref[...], n3b_ref[...])
    out_ref[...] = x3


def _tc_post(x, xl, xr, ei2d, agg_p, asum_p, we_row, att_row, gat_bias,
             n1w, n1b, W2, b2, n2w, n2b, W3, b3, n3w, n3b):
    return pl.pallas_call(
        _post_body,
        out_shape=jax.ShapeDtypeStruct((V, D), jnp.float32),
    )(x, xl, xr, ei2d, agg_p, asum_p, we_row, att_row, gat_bias,
      n1w, n1b, W2, b2, n2w, n2b, W3, b3, n3w, n3b)


@jax.jit
def kernel(x, full_edge_index, edge_indicators, batch, spectrum_mats, Ws, bs,
           Wl, bl, Wr, br, We, att, gat_bias, n1w, n1b, W2, b2, n2w, n2b,
           W3, b3, n3w, n3b):
    mat = spectrum_mats[0]
    xl, xr = _tc_pre(x, mat, Ws, bs, Wl[:D], Wl[D:], bl, Wr[:D], Wr[D:], br)

    src2d = full_edge_index[0].reshape(NW * NCHUNK, EK)
    dst2d = full_edge_index[1].reshape(NW * NCHUNK, EK)
    ea2d = edge_indicators.reshape(NW * NCHUNK, EK)
    we_flat = We.reshape(HC)
    att_flat = att.reshape(HC)

    def _ilv(t):
        n = t.shape[0]
        return (t.reshape(n, H, 2, LANES).transpose(0, 1, 3, 2)
                .reshape(n, HC).astype(jnp.bfloat16))

    agg_p, asum_p = _sc_edge(_ilv(xl), _ilv(xr), src2d, dst2d, ea2d,
                             _ilv(we_flat.reshape(1, HC)).reshape(HC),
                             _ilv(att_flat.reshape(1, HC)).reshape(HC))
    agg_p = agg_p.reshape(NC, V, HC)
    asum_p = asum_p.reshape(NC, V, LANES)

    ei2d = edge_indicators.reshape(V, E // V)
    return _tc_post(x, xl, xr, ei2d, agg_p, asum_p, We.reshape(1, HC),
                    att.reshape(1, HC), gat_bias, n1w, n1b, W2, b2, n2w,
                    n2b, W3, b3, n3w, n3b)
